# baseline (speedup 1.0000x reference)
import jax
import jax.numpy as jnp
from jax.experimental import pallas as pl
from jax.experimental.pallas import tpu as pltpu

B, S, D = 4, 4096, 4096


def _pool_body(mask_ref, hid_ref, out_ref, rows_ref, gsems, wsems):
    totals = jnp.sum(mask_ref[...], axis=1)
    lp = jnp.sum(mask_ref[:, pl.ds(S - 1, 1)])
    gathers = []
    for b in range(B):
        idx = jnp.where(lp == B, S - 1, (totals[b] - 1) & (S - 1))
        gathers.append(pltpu.make_async_copy(
            hid_ref.at[b, idx], rows_ref.at[b], gsems.at[b]))
    for g in gathers:
        g.start()
    writes = []
    for b in range(B):
        gathers[b].wait()
        w = pltpu.make_async_copy(rows_ref.at[b], out_ref.at[b], wsems.at[b])
        w.start()
        writes.append(w)
    for w in writes:
        w.wait()


def kernel(last_hidden_state, attention_mask):
    return pl.pallas_call(
        _pool_body,
        in_specs=[
            pl.BlockSpec((B, S), lambda: (0, 0)),
            pl.BlockSpec(memory_space=pl.ANY),
        ],
        out_specs=pl.BlockSpec(memory_space=pl.ANY),
        out_shape=jax.ShapeDtypeStruct((B, D), jnp.float32),
        scratch_shapes=[pltpu.VMEM((B, D), jnp.float32),
                        pltpu.SemaphoreType.DMA((B,)),
                        pltpu.SemaphoreType.DMA((B,))],
    )(attention_mask.astype(jnp.int32), last_hidden_state)
